# baseline (speedup 1.0000x reference)
import functools

import jax
import jax.numpy as jnp
from jax import lax
from jax.experimental import pallas as pl
from jax.experimental.pallas import tpu as pltpu
from jax.experimental.pallas import tpu_sc as plsc

B, T, C = 1, 2048, 768
E = 8
K = 2
H = 1024
N = B * T
NK = N * K
BM = 256
NPAD = NK + E * BM
NBLK = NPAD // BM

NW = 32
TPW = N // NW
SUB = 32
LANES = 16


def _router_body(x_ref, wr_ref, br_ref, logits_ref, idx_ref, aux_ref,
                 meta_ref, sbc_ref):
    x = x_ref[...]
    lg = jnp.dot(x, wr_ref[...], preferred_element_type=jnp.float32)
    lg = lg + br_ref[...]
    logits_ref[...] = lg
    lane = lax.broadcasted_iota(jnp.int32, (T, E), 1)
    neg = jnp.float32(-1e30)
    v0 = jnp.max(lg, axis=1, keepdims=True)
    i0 = jnp.argmax(lg, axis=1).reshape(T, 1)
    lg2 = jnp.where(lane == i0, neg, lg)
    v1 = jnp.max(lg2, axis=1, keepdims=True)
    i1 = jnp.argmax(lg2, axis=1).reshape(T, 1)
    e1 = jnp.exp(v1 - v0)
    s0 = 1.0 / (1.0 + e1)
    s1 = 1.0 - s0
    klane = lax.broadcasted_iota(jnp.int32, (T, K), 1)
    idx_ref[...] = jnp.where(klane == 0, i0, i1)
    slane = lax.broadcasted_iota(jnp.int32, (T, 2 * LANES), 1)
    sbc_ref[...] = jnp.where(slane < LANES, s0, s1)

    c0 = (lane == i0).astype(jnp.float32)
    c1 = (lane == i1).astype(jnp.float32)
    s = c0 + c1
    inc = s
    k = 1
    while k < T:
        shifted = jnp.concatenate(
            [jnp.zeros((k, E), jnp.float32), inc[: T - k, :]], axis=0)
        inc = inc + shifted
        k *= 2
    ex = inc - s
    counts = jnp.sum(s, axis=0, keepdims=True)
    pc = jnp.floor((counts + (BM - 1)) / BM) * BM
    triu = (lax.broadcasted_iota(jnp.int32, (E, E), 0) <
            lax.broadcasted_iota(jnp.int32, (E, E), 1)).astype(jnp.float32)
    pstart = jnp.dot(pc, triu, preferred_element_type=jnp.float32)
    cum_pc = pstart + pc
    tot = jnp.sum(pc, axis=1, keepdims=True)

    rank0 = jnp.sum(c0 * ex, axis=1, keepdims=True)
    rank1 = jnp.sum(c1 * (ex + c0), axis=1, keepdims=True)
    base0 = jnp.sum(c0 * pstart, axis=1, keepdims=True)
    base1 = jnp.sum(c1 * pstart, axis=1, keepdims=True)
    pos0 = base0 + rank0
    pos1 = base1 + rank1
    plane = lax.broadcasted_iota(jnp.int32, (T, K), 1)
    aux_ref[...] = jnp.where(plane == 0, pos0, pos1).astype(jnp.int32)

    jgrid = lax.broadcasted_iota(jnp.int32, (128, E), 0) * BM
    cum_i = jnp.broadcast_to(cum_pc.astype(jnp.int32), (128, E))
    be = jnp.sum((cum_i <= jgrid).astype(jnp.int32), axis=1, keepdims=True)
    be = jnp.minimum(be, E - 1)
    act = (jgrid[:, :1] < tot.astype(jnp.int32)).astype(jnp.int32)
    par = jnp.bitwise_and(be, 1)
    be_e = jnp.minimum(be + par, E - 2)
    be_o = jnp.minimum(be + 1 - par, E - 1)
    mlane = lax.broadcasted_iota(jnp.int32, (128, E), 1)
    meta_ref[...] = jnp.where(
        mlane == 0, be,
        jnp.where(mlane == 1, act, jnp.where(mlane == 2, be_e, be_o)))


def _router(x2d, wr, br2d):
    return pl.pallas_call(
        _router_body,
        out_shape=(
            jax.ShapeDtypeStruct((T, E), jnp.float32),
            jax.ShapeDtypeStruct((T, K), jnp.int32),
            jax.ShapeDtypeStruct((T, K), jnp.int32),
            jax.ShapeDtypeStruct((128, E), jnp.int32),
            jax.ShapeDtypeStruct((T, 2 * LANES), jnp.float32),
        ),
    )(x2d, wr, br2d)


def _dispatch_body(x_hbm, post_hbm, xs_hbm, idx0_v, idx1_v, rows_v, sem):
    wid = lax.axis_index("s") * 2 + lax.axis_index("c")
    base = wid * TPW
    cp = pltpu.async_copy(x_hbm.at[pl.ds(base, TPW)], rows_v, sem)
    pltpu.sync_copy(post_hbm.at[0, pl.ds(base, TPW)], idx0_v)
    pltpu.sync_copy(post_hbm.at[1, pl.ds(base, TPW)], idx1_v)
    cp.wait()
    s0 = pltpu.async_copy(rows_v, xs_hbm.at[idx0_v], sem)
    s1 = pltpu.async_copy(rows_v, xs_hbm.at[idx1_v], sem)
    s0.wait()
    s1.wait()


def _dispatch(x2d, post):
    mesh = plsc.VectorSubcoreMesh(core_axis_name="c", subcore_axis_name="s")
    return pl.kernel(
        _dispatch_body,
        out_type=jax.ShapeDtypeStruct((NPAD, C), jnp.float32),
        mesh=mesh,
        scratch_types=[
            pltpu.VMEM((TPW,), jnp.int32),
            pltpu.VMEM((TPW,), jnp.int32),
            pltpu.VMEM((TPW, C), jnp.float32),
            pltpu.SemaphoreType.DMA,
        ],
    )(x2d, post)


def _ffn_body(meta_ref, xs_ref, w1_ref, b1_ref, w2_ref, b2_ref, ys_ref,
              w1b_s, w2b_s, ce_s):
    j = pl.program_id(0)

    @pl.when(j == 0)
    def _():
        ce_s[0] = -1

    e = meta_ref[j, 0]
    active = meta_ref[j, 1] == 1

    @pl.when(jnp.logical_and(active, ce_s[0] != e))
    def _():
        w1b_s[...] = w1_ref[0].astype(jnp.bfloat16)
        w2b_s[...] = w2_ref[0].astype(jnp.bfloat16)
        ce_s[0] = e

    @pl.when(active)
    def _():
        xb = xs_ref[...].astype(jnp.bfloat16)
        h = jnp.dot(xb, w1b_s[...], preferred_element_type=jnp.float32)
        h = jnp.maximum(h + b1_ref[0], 0.0).astype(jnp.bfloat16)
        y = jnp.dot(h, w2b_s[...], preferred_element_type=jnp.float32)
        ys_ref[...] = y + b2_ref[0]


def _ffn(meta, xs_pad, w1, b1, w2, b2):
    grid_spec = pltpu.PrefetchScalarGridSpec(
        num_scalar_prefetch=1,
        grid=(NBLK,),
        in_specs=[
            pl.BlockSpec((BM, C), lambda j, m: (j, 0)),
            pl.BlockSpec((1, C, H), lambda j, m: (m[j, 0], 0, 0)),
            pl.BlockSpec((1, 1, H), lambda j, m: (m[j, 0], 0, 0)),
            pl.BlockSpec((1, H, C), lambda j, m: (m[j, 0], 0, 0)),
            pl.BlockSpec((1, 1, C), lambda j, m: (m[j, 0], 0, 0)),
        ],
        out_specs=pl.BlockSpec((BM, C), lambda j, m: (j, 0)),
        scratch_shapes=[
            pltpu.VMEM((C, H), jnp.bfloat16),
            pltpu.VMEM((H, C), jnp.bfloat16),
            pltpu.SMEM((1,), jnp.int32),
        ],
    )
    return pl.pallas_call(
        _ffn_body,
        grid_spec=grid_spec,
        out_shape=jax.ShapeDtypeStruct((NPAD, C), jnp.float32),
    )(meta, xs_pad, w1, b1, w2, b2)


def _combine_body(ys_hbm, post_hbm, sbc_hbm, out_hbm, sb_v, idx0_v, idx1_v,
                  r0_v, r1_v, o_v, sem):
    wid = lax.axis_index("s") * 2 + lax.axis_index("c")
    for p in range(TPW // SUB):
        base = wid * SUB + p * (NW * SUB)
        pltpu.sync_copy(post_hbm.at[0, pl.ds(base, SUB)], idx0_v)
        pltpu.sync_copy(post_hbm.at[1, pl.ds(base, SUB)], idx1_v)
        pltpu.sync_copy(sbc_hbm.at[pl.ds(base, SUB)], sb_v)
        cp0 = pltpu.async_copy(ys_hbm.at[idx0_v], r0_v, sem)
        cp1 = pltpu.async_copy(ys_hbm.at[idx1_v], r1_v, sem)
        cp0.wait()
        cp1.wait()

        def tok(r, carry):
            s0 = sb_v[r, pl.ds(0, LANES)]
            s1 = sb_v[r, pl.ds(LANES, LANES)]
            for cc in range(C // LANES):
                a = r0_v[r, pl.ds(cc * LANES, LANES)]
                b = r1_v[r, pl.ds(cc * LANES, LANES)]
                o_v[r, pl.ds(cc * LANES, LANES)] = a * s0 + b * s1
            return carry

        lax.fori_loop(0, SUB, tok, 0)
        pltpu.sync_copy(o_v, out_hbm.at[pl.ds(base, SUB)])


def _combine(ys, post, sbc):
    mesh = plsc.VectorSubcoreMesh(core_axis_name="c", subcore_axis_name="s")
    return pl.kernel(
        _combine_body,
        out_type=jax.ShapeDtypeStruct((N, C), jnp.float32),
        mesh=mesh,
        scratch_types=[
            pltpu.VMEM((SUB, 2 * LANES), jnp.float32),
            pltpu.VMEM((SUB,), jnp.int32),
            pltpu.VMEM((SUB,), jnp.int32),
            pltpu.VMEM((SUB, C), jnp.float32),
            pltpu.VMEM((SUB, C), jnp.float32),
            pltpu.VMEM((SUB, C), jnp.float32),
            pltpu.SemaphoreType.DMA,
        ],
    )(ys, post, sbc)


@jax.jit
def kernel(x, Wr, br, W1, b1, W2, b2):
    x2d = x.reshape(N, C)
    logits, idx2, pos2, meta, sbc = _router(x2d, Wr, br.reshape(1, E))
    post = pos2.T
    xs_pad = _dispatch(x2d, post)
    ys = _ffn(meta, xs_pad, W1, b1.reshape(E, 1, H), W2, b2.reshape(E, 1, C))
    out2d = _combine(ys, post, sbc)
    return (logits.reshape(B, T, E), idx2.reshape(B, T, K),
            out2d.reshape(B, T, C))


# speedup vs baseline: 1.0545x; 1.0545x over previous
import functools

import jax
import jax.numpy as jnp
from jax import lax
from jax.experimental import pallas as pl
from jax.experimental.pallas import tpu as pltpu
from jax.experimental.pallas import tpu_sc as plsc

B, T, C = 1, 2048, 768
E = 8
K = 2
H = 1024
N = B * T
NK = N * K
BM = 256
NPAD = NK + E * BM
NBLK = NPAD // BM

NW = 32
TPW = N // NW
SUB = 32
LANES = 16


def _router_body(x_ref, wr_ref, br_ref, logits_ref, idx_ref, aux_ref,
                 meta_ref, sbc_ref):
    x = x_ref[...]
    lg = jnp.dot(x, wr_ref[...], preferred_element_type=jnp.float32)
    lg = lg + br_ref[...]
    logits_ref[...] = lg
    lane = lax.broadcasted_iota(jnp.int32, (T, E), 1)
    neg = jnp.float32(-1e30)
    v0 = jnp.max(lg, axis=1, keepdims=True)
    i0 = jnp.argmax(lg, axis=1).reshape(T, 1)
    lg2 = jnp.where(lane == i0, neg, lg)
    v1 = jnp.max(lg2, axis=1, keepdims=True)
    i1 = jnp.argmax(lg2, axis=1).reshape(T, 1)
    e1 = jnp.exp(v1 - v0)
    s0 = 1.0 / (1.0 + e1)
    s1 = 1.0 - s0
    klane = lax.broadcasted_iota(jnp.int32, (T, K), 1)
    idx_ref[...] = jnp.where(klane == 0, i0, i1)
    slane = lax.broadcasted_iota(jnp.int32, (T, 2 * LANES), 1)
    sbc_ref[...] = jnp.where(slane < LANES, s0, s1)

    c0 = (lane == i0).astype(jnp.float32)
    c1 = (lane == i1).astype(jnp.float32)
    s = c0 + c1
    inc = s
    k = 1
    while k < T:
        shifted = jnp.concatenate(
            [jnp.zeros((k, E), jnp.float32), inc[: T - k, :]], axis=0)
        inc = inc + shifted
        k *= 2
    ex = inc - s
    counts = jnp.sum(s, axis=0, keepdims=True)
    pc = jnp.floor((counts + (BM - 1)) / BM) * BM
    triu = (lax.broadcasted_iota(jnp.int32, (E, E), 0) <
            lax.broadcasted_iota(jnp.int32, (E, E), 1)).astype(jnp.float32)
    pstart = jnp.dot(pc, triu, preferred_element_type=jnp.float32)
    cum_pc = pstart + pc
    tot = jnp.sum(pc, axis=1, keepdims=True)

    rank0 = jnp.sum(c0 * ex, axis=1, keepdims=True)
    rank1 = jnp.sum(c1 * (ex + c0), axis=1, keepdims=True)
    base0 = jnp.sum(c0 * pstart, axis=1, keepdims=True)
    base1 = jnp.sum(c1 * pstart, axis=1, keepdims=True)
    pos0 = base0 + rank0
    pos1 = base1 + rank1
    plane = lax.broadcasted_iota(jnp.int32, (T, K), 1)
    aux_ref[...] = jnp.where(plane == 0, pos0, pos1).astype(jnp.int32)

    jgrid = lax.broadcasted_iota(jnp.int32, (128, E), 0) * BM
    cum_i = jnp.broadcast_to(cum_pc.astype(jnp.int32), (128, E))
    be = jnp.sum((cum_i <= jgrid).astype(jnp.int32), axis=1, keepdims=True)
    be = jnp.minimum(be, E - 1)
    act = (jgrid[:, :1] < tot.astype(jnp.int32)).astype(jnp.int32)
    par = jnp.bitwise_and(be, 1)
    be_e = jnp.minimum(be + par, E - 2)
    be_o = jnp.minimum(be + 1 - par, E - 1)
    mlane = lax.broadcasted_iota(jnp.int32, (128, E), 1)
    meta_ref[...] = jnp.where(
        mlane == 0, be,
        jnp.where(mlane == 1, act, jnp.where(mlane == 2, be_e, be_o)))


def _router(x2d, wr, br2d):
    return pl.pallas_call(
        _router_body,
        out_shape=(
            jax.ShapeDtypeStruct((T, E), jnp.float32),
            jax.ShapeDtypeStruct((T, K), jnp.int32),
            jax.ShapeDtypeStruct((T, K), jnp.int32),
            jax.ShapeDtypeStruct((128, E), jnp.int32),
            jax.ShapeDtypeStruct((T, 2 * LANES), jnp.float32),
        ),
    )(x2d, wr, br2d)


def _dispatch_body(x_hbm, post_hbm, xs_hbm, idx0_v, idx1_v, rows_v, sem):
    wid = lax.axis_index("s") * 2 + lax.axis_index("c")
    base = wid * TPW
    cp = pltpu.async_copy(x_hbm.at[pl.ds(base, TPW)], rows_v, sem)
    pltpu.sync_copy(post_hbm.at[0, pl.ds(base, TPW)], idx0_v)
    pltpu.sync_copy(post_hbm.at[1, pl.ds(base, TPW)], idx1_v)
    cp.wait()
    s0 = pltpu.async_copy(rows_v, xs_hbm.at[idx0_v], sem)
    s1 = pltpu.async_copy(rows_v, xs_hbm.at[idx1_v], sem)
    s0.wait()
    s1.wait()


def _dispatch(x2d, post):
    mesh = plsc.VectorSubcoreMesh(core_axis_name="c", subcore_axis_name="s")
    return pl.kernel(
        _dispatch_body,
        out_type=jax.ShapeDtypeStruct((NPAD, C), jnp.float32),
        mesh=mesh,
        scratch_types=[
            pltpu.VMEM((TPW,), jnp.int32),
            pltpu.VMEM((TPW,), jnp.int32),
            pltpu.VMEM((TPW, C), jnp.float32),
            pltpu.SemaphoreType.DMA,
        ],
    )(x2d, post)


def _ffn_body(meta_ref, xs_ref, w1e_ref, w1o_ref, b1_ref, w2e_ref, w2o_ref,
              b2_ref, ys_ref):
    j = pl.program_id(0)
    active = meta_ref[j, 1] == 1
    par = jnp.bitwise_and(meta_ref[j, 0], 1)

    def compute(w1_ref, w2_ref):
        xb = xs_ref[...].astype(jnp.bfloat16)
        w1 = w1_ref[0].astype(jnp.bfloat16)
        h = jnp.dot(xb, w1, preferred_element_type=jnp.float32)
        h = jnp.maximum(h + b1_ref[0], 0.0).astype(jnp.bfloat16)
        w2 = w2_ref[0].astype(jnp.bfloat16)
        y = jnp.dot(h, w2, preferred_element_type=jnp.float32)
        ys_ref[...] = y + b2_ref[0]

    @pl.when(jnp.logical_and(active, par == 0))
    def _():
        compute(w1e_ref, w2e_ref)

    @pl.when(jnp.logical_and(active, par == 1))
    def _():
        compute(w1o_ref, w2o_ref)


def _ffn(meta, xs_pad, w1, b1, w2, b2):
    grid_spec = pltpu.PrefetchScalarGridSpec(
        num_scalar_prefetch=1,
        grid=(NBLK,),
        in_specs=[
            pl.BlockSpec((BM, C), lambda j, m: (j, 0)),
            pl.BlockSpec((1, C, H), lambda j, m: (m[j, 2], 0, 0)),
            pl.BlockSpec((1, C, H), lambda j, m: (m[j, 3], 0, 0)),
            pl.BlockSpec((1, 1, H), lambda j, m: (m[j, 0], 0, 0)),
            pl.BlockSpec((1, H, C), lambda j, m: (m[j, 2], 0, 0)),
            pl.BlockSpec((1, H, C), lambda j, m: (m[j, 3], 0, 0)),
            pl.BlockSpec((1, 1, C), lambda j, m: (m[j, 0], 0, 0)),
        ],
        out_specs=pl.BlockSpec((BM, C), lambda j, m: (j, 0)),
    )
    return pl.pallas_call(
        _ffn_body,
        grid_spec=grid_spec,
        out_shape=jax.ShapeDtypeStruct((NPAD, C), jnp.float32),
    )(meta, xs_pad, w1, w1, b1, w2, w2, b2)


def _combine_body(ys_hbm, post_hbm, sbc_hbm, out_hbm, sb_v, idx0_v, idx1_v,
                  r0_v, r1_v, sem):
    wid = lax.axis_index("s") * 2 + lax.axis_index("c")
    base = wid * TPW
    pltpu.sync_copy(post_hbm.at[0, pl.ds(base, TPW)], idx0_v)
    pltpu.sync_copy(post_hbm.at[1, pl.ds(base, TPW)], idx1_v)
    cp0 = pltpu.async_copy(ys_hbm.at[idx0_v], r0_v, sem)
    cp1 = pltpu.async_copy(ys_hbm.at[idx1_v], r1_v, sem)
    pltpu.sync_copy(sbc_hbm.at[pl.ds(base, TPW)], sb_v)
    cp0.wait()
    cp1.wait()

    def tok(r, carry):
        s0 = sb_v[r, pl.ds(0, LANES)]
        s1 = sb_v[r, pl.ds(LANES, LANES)]
        for cc in range(C // LANES):
            a = r0_v[r, pl.ds(cc * LANES, LANES)]
            b = r1_v[r, pl.ds(cc * LANES, LANES)]
            r0_v[r, pl.ds(cc * LANES, LANES)] = a * s0 + b * s1
        return carry

    lax.fori_loop(0, TPW, tok, 0)
    pltpu.sync_copy(r0_v, out_hbm.at[pl.ds(base, TPW)])


def _combine(ys, post, sbc):
    mesh = plsc.VectorSubcoreMesh(core_axis_name="c", subcore_axis_name="s")
    return pl.kernel(
        _combine_body,
        out_type=jax.ShapeDtypeStruct((N, C), jnp.float32),
        mesh=mesh,
        scratch_types=[
            pltpu.VMEM((TPW, 2 * LANES), jnp.float32),
            pltpu.VMEM((TPW,), jnp.int32),
            pltpu.VMEM((TPW,), jnp.int32),
            pltpu.VMEM((TPW, C), jnp.float32),
            pltpu.VMEM((TPW, C), jnp.float32),
            pltpu.SemaphoreType.DMA,
        ],
    )(ys, post, sbc)


@jax.jit
def kernel(x, Wr, br, W1, b1, W2, b2):
    x2d = x.reshape(N, C)
    logits, idx2, pos2, meta, sbc = _router(x2d, Wr, br.reshape(1, E))
    post = pos2.T
    xs_pad = _dispatch(x2d, post)
    ys = _ffn(meta, xs_pad, W1, b1.reshape(E, 1, H), W2, b2.reshape(E, 1, C))
    out2d = _combine(ys, post, sbc)
    return (logits.reshape(B, T, E), idx2.reshape(B, T, K),
            out2d.reshape(B, T, C))
